# Initial kernel scaffold; baseline (speedup 1.0000x reference)
import jax
import jax.numpy as jnp
from jax.experimental import pallas as pl


def kernel(x, k, W_g, W_noise):
    raise NotImplementedError("write your pallas kernel here")


# speedup vs baseline: 1.0131x; 1.0131x over previous
import functools

import jax
import jax.numpy as jnp
from jax.experimental import pallas as pl
from jax.experimental.pallas import tpu as pltpu

_B, _T, _E, _NE = 4, 2048, 2048, 64
_M = _B * _T
_BLK = 512
_KMAX = 8


def _router_kernel(k_ref, x_ref, w_ref, z_ref, o_ref):
    k = k_ref[0]
    h2 = jnp.dot(x_ref[...], w_ref[...], preferred_element_type=jnp.float32)
    prelim = h2[:, :_NE]
    noise = h2[:, _NE:]
    sp = jnp.maximum(noise, 0.0) + jnp.log1p(jnp.exp(-jnp.abs(noise)))
    h = prelim + z_ref[...] + sp

    work = h
    removed = jnp.zeros((h.shape[0], 1), jnp.int32)
    done = jnp.zeros((h.shape[0], 1), jnp.bool_)
    thr = jnp.full((h.shape[0], 1), -jnp.inf, jnp.float32)
    row_max = jnp.max(h, axis=1, keepdims=True)
    for _ in range(_KMAX):
        m = jnp.max(work, axis=1, keepdims=True)
        eq = work == m
        c = jnp.sum(eq.astype(jnp.int32), axis=1, keepdims=True)
        thr = jnp.where(done, thr, m)
        done = jnp.logical_or(done, removed + c >= k)
        removed = removed + c
        work = jnp.where(eq, -jnp.inf, work)

    keep = h >= thr
    e = jnp.where(keep, jnp.exp(h - row_max), 0.0)
    o_ref[...] = e / jnp.sum(e, axis=1, keepdims=True)


def kernel(x, k, W_g, W_noise):
    xm = x.reshape(_M, _E)
    w = jnp.concatenate([W_g, W_noise], axis=1)
    z = jax.random.normal(jax.random.key(42), (_B, _T, _NE), dtype=jnp.float32)
    zm = z.reshape(_M, _NE)
    ks = jnp.asarray(k, jnp.int32).reshape(1)

    out = pl.pallas_call(
        _router_kernel,
        grid=(_M // _BLK,),
        in_specs=[
            pl.BlockSpec(memory_space=pltpu.SMEM),
            pl.BlockSpec((_BLK, _E), lambda i: (i, 0)),
            pl.BlockSpec((_E, 2 * _NE), lambda i: (0, 0)),
            pl.BlockSpec((_BLK, _NE), lambda i: (i, 0)),
        ],
        out_specs=pl.BlockSpec((_BLK, _NE), lambda i: (i, 0)),
        out_shape=jax.ShapeDtypeStruct((_M, _NE), jnp.float32),
    )(ks, xm, w, zm)
    return out.reshape(_B, _T, _NE)
